# baseline (speedup 1.0000x reference)
import functools

import jax
import jax.numpy as jnp
from jax.experimental import pallas as pl

_NB = 8


def _reduce_body(x_ref, o_ref, *, scale):
    o_ref[...] = jnp.sum(x_ref[...], axis=1, keepdims=True) * scale


def kernel(x, protos):
    N, M, C, T, V = x.shape
    P = protos.shape[0]
    scale = 1.0 / (M * P)
    rows = M * V * T

    xt = jnp.transpose(x, (0, 1, 4, 3, 2)).reshape(N, rows, C)

    out = pl.pallas_call(
        functools.partial(_reduce_body, scale=scale),
        grid=(N // _NB,),
        in_specs=[pl.BlockSpec((_NB, rows, C), lambda i: (i, 0, 0))],
        out_specs=pl.BlockSpec((_NB, 1, C), lambda i: (i, 0, 0)),
        out_shape=jax.ShapeDtypeStruct((N, 1, C), x.dtype),
    )(xt)
    return out.reshape(N, C)


# speedup vs baseline: 1.0311x; 1.0311x over previous
import functools

import jax
import jax.numpy as jnp
from jax.experimental import pallas as pl

_NB = 2


def _reduce_body(xa_ref, xb_ref, oa_ref, ob_ref, *, scale):
    oa_ref[...] = jnp.sum(xa_ref[...], axis=1, keepdims=True) * scale
    ob_ref[...] = jnp.sum(xb_ref[...], axis=1, keepdims=True) * scale


def kernel(x, protos):
    N, M, C, T, V = x.shape
    P = protos.shape[0]
    scale = 1.0 / (M * P)
    rows = M * V * T
    half = N // 2
    steps = half // _NB

    xt = jnp.transpose(x, (0, 1, 4, 3, 2)).reshape(N, rows, C)

    out_a, out_b = pl.pallas_call(
        functools.partial(_reduce_body, scale=scale),
        grid=(steps,),
        in_specs=[
            pl.BlockSpec((_NB, rows, C), lambda i: (i, 0, 0)),
            pl.BlockSpec((_NB, rows, C), lambda i: (i + steps, 0, 0)),
        ],
        out_specs=[
            pl.BlockSpec((_NB, 1, C), lambda i: (i, 0, 0)),
            pl.BlockSpec((_NB, 1, C), lambda i: (i, 0, 0)),
        ],
        out_shape=[
            jax.ShapeDtypeStruct((half, 1, C), x.dtype),
            jax.ShapeDtypeStruct((half, 1, C), x.dtype),
        ],
    )(xt, xt)
    return jnp.concatenate([out_a, out_b], axis=0).reshape(N, C)
